# baseline (speedup 1.0000x reference)
import functools

import jax
import jax.numpy as jnp
from jax import lax
from jax.experimental import pallas as pl
from jax.experimental.pallas import tpu as pltpu
from jax.experimental.pallas import tpu_sc as plsc

NC = 2
NS = 16
NW = NC * NS
SEQ = 16384
D = 16
NWORDS = 1000000
PER_W = SEQ // NW
HB = 1 << 20
STRIPE = HB // NS
ZCH = 4096
CHUNK = 128
NCHUNK = PER_W // CHUNK

C = 65536
NBLK = (NWORDS + C - 1) // C


def _sc_histogram(idx):
    mesh = plsc.VectorSubcoreMesh(core_axis_name="c", subcore_axis_name="s")

    @functools.partial(
        pl.kernel,
        mesh=mesh,
        out_type=jax.ShapeDtypeStruct((NC * HB,), jnp.float32),
        scratch_types=[
            pltpu.VMEM((NCHUNK, CHUNK), jnp.int32),
            pltpu.VMEM((CHUNK,), jnp.float32),
            pltpu.VMEM((ZCH,), jnp.float32),
            pltpu.VMEM_SHARED((HB,), jnp.float32),
            pltpu.SemaphoreType.DMA,
        ],
    )
    def body(idx_hbm, out_hbm, idx2_v, ones_v, z_v, hsh, sem):
        c = lax.axis_index("c")
        s = lax.axis_index("s")
        wid = s * NC + c
        for k in range(NCHUNK):
            pltpu.sync_copy(
                idx_hbm.at[pl.ds(wid * PER_W + k * CHUNK, CHUNK)], idx2_v.at[k]
            )

        one = jnp.ones((16,), jnp.float32)
        zero = jnp.zeros((16,), jnp.float32)
        for z in range(ZCH // 16):
            z_v[pl.ds(z * 16, 16)] = zero
        for z in range(CHUNK // 16):
            ones_v[pl.ds(z * 16, 16)] = one
        for z in range(STRIPE // ZCH):
            pltpu.sync_copy(z_v, hsh.at[pl.ds(s * STRIPE + z * ZCH, ZCH)])

        plsc.subcore_barrier()
        for k in range(NCHUNK):
            pltpu.sync_copy(ones_v, hsh.at[idx2_v.at[k]], add=True)
        plsc.subcore_barrier()

        pltpu.sync_copy(
            hsh.at[pl.ds(s * STRIPE, STRIPE)],
            out_hbm.at[pl.ds(c * HB + s * STRIPE, STRIPE)],
        )

    return body(idx)


def _tc_matvec_finalize(emb_t, hist, bias2d):
    def body(t_ref, h_ref, b_ref, o_ref, acc_ref, hs_ref):
        i = pl.program_id(0)

        @pl.when(i == 0)
        def _():
            acc_ref[...] = jnp.zeros_like(acc_ref)
            for q in range(HB // 8192):
                sl = pl.ds(q * 8192, 8192)
                sl1 = pl.ds(HB + q * 8192, 8192)
                hs_ref[sl] = h_ref[sl] + h_ref[sl1]

        def chunk_sum(mask_from):
            accs = [jnp.zeros((D, 128), jnp.float32) for _ in range(8)]
            lane = lax.broadcasted_iota(jnp.int32, (1, 128), 1)
            for k in range(C // 128):
                sl = pl.ds(k * 128, 128)
                prod = t_ref[:, sl] * hs_ref[pl.ds(i * C + k * 128, 128)][None, :]
                if mask_from is not None and k >= mask_from:
                    col = i * C + k * 128 + lane
                    prod = jnp.where(col < NWORDS, prod, 0.0)
                accs[k % 8] = accs[k % 8] + prod
            while len(accs) > 1:
                accs = [a + b for a, b in zip(accs[::2], accs[1::2])]
            return accs[0]

        @pl.when(i < NBLK - 1)
        def _():
            acc_ref[...] += chunk_sum(None)

        @pl.when(i == NBLK - 1)
        def _():
            part = chunk_sum((NWORDS - (NBLK - 1) * C) // 128)
            score = jnp.sum(acc_ref[...] + part, axis=1)[None, :] + b_ref[...]
            m = jnp.max(score, axis=1, keepdims=True)
            lse = jnp.log(jnp.sum(jnp.exp(score - m), axis=1, keepdims=True)) + m
            o_ref[...] = score - lse

    return pl.pallas_call(
        body,
        grid=(NBLK,),
        in_specs=[
            pl.BlockSpec((D, C), lambda i: (0, i)),
            pl.BlockSpec((NC * HB,), lambda i: (0,)),
            pl.BlockSpec((1, D), lambda i: (0, 0)),
        ],
        out_specs=pl.BlockSpec((1, D), lambda i: (0, 0)),
        out_shape=jax.ShapeDtypeStruct((1, D), jnp.float32),
        scratch_shapes=[
            pltpu.VMEM((D, 128), jnp.float32),
            pltpu.VMEM((HB,), jnp.float32),
        ],
    )(emb_t, hist, bias2d)


def kernel(inputs, embeddings, bias):
    idx = inputs.astype(jnp.int32)
    emb_t = embeddings.T
    hist = _sc_histogram(idx)
    return _tc_matvec_finalize(emb_t, hist, bias.reshape(1, D))


# speedup vs baseline: 1.0714x; 1.0714x over previous
import functools

import jax
import jax.numpy as jnp
from jax import lax
from jax.experimental import pallas as pl
from jax.experimental.pallas import tpu as pltpu
from jax.experimental.pallas import tpu_sc as plsc

NC = 2
NS = 16
NW = NC * NS
SEQ = 16384
D = 16
NWORDS = 1000000
PER_W = SEQ // NW
HB = 1 << 20
STRIPE = HB // NS
ZCH = 4096
CHUNK = 128
NCHUNK = PER_W // CHUNK

C = 65536
NBLK = (NWORDS + C - 1) // C


def _sc_histogram(idx):
    mesh = plsc.VectorSubcoreMesh(core_axis_name="c", subcore_axis_name="s")

    @functools.partial(
        pl.kernel,
        mesh=mesh,
        out_type=jax.ShapeDtypeStruct((NC * HB,), jnp.float32),
        scratch_types=[
            pltpu.VMEM((NCHUNK, CHUNK), jnp.int32),
            pltpu.VMEM((CHUNK,), jnp.float32),
            pltpu.VMEM((ZCH,), jnp.float32),
            pltpu.VMEM_SHARED((HB,), jnp.float32),
            pltpu.SemaphoreType.DMA,
        ],
    )
    def body(idx_hbm, out_hbm, idx2_v, ones_v, z_v, hsh, sem):
        c = lax.axis_index("c")
        s = lax.axis_index("s")
        wid = s * NC + c
        for k in range(NCHUNK):
            pltpu.sync_copy(
                idx_hbm.at[pl.ds(wid * PER_W + k * CHUNK, CHUNK)], idx2_v.at[k]
            )

        one = jnp.ones((16,), jnp.float32)
        zero = jnp.zeros((16,), jnp.float32)
        for z in range(ZCH // 16):
            z_v[pl.ds(z * 16, 16)] = zero
        for z in range(CHUNK // 16):
            ones_v[pl.ds(z * 16, 16)] = one
        for z in range(STRIPE // ZCH):
            pltpu.sync_copy(z_v, hsh.at[pl.ds(s * STRIPE + z * ZCH, ZCH)])

        plsc.subcore_barrier()
        for k in range(NCHUNK):
            pltpu.sync_copy(ones_v, hsh.at[idx2_v.at[k]], add=True)
        plsc.subcore_barrier()

        pltpu.sync_copy(
            hsh.at[pl.ds(s * STRIPE, STRIPE)],
            out_hbm.at[pl.ds(c * HB + s * STRIPE, STRIPE)],
        )

    return body(idx)


def _tc_matvec_finalize(emb_t, hist, bias2d):
    def body(t_ref, h0_ref, h1_ref, b_ref, o_ref, acc_ref):
        i = pl.program_id(0)

        @pl.when(i == 0)
        def _():
            acc_ref[...] = jnp.zeros_like(acc_ref)

        def chunk_sum(mask_from):
            accs = [jnp.zeros((D, 128), jnp.float32) for _ in range(8)]
            lane = lax.broadcasted_iota(jnp.int32, (1, 128), 1)
            for k in range(C // 128):
                sl = pl.ds(k * 128, 128)
                prod = t_ref[:, sl] * (h0_ref[sl] + h1_ref[sl])[None, :]
                if mask_from is not None and k >= mask_from:
                    col = i * C + k * 128 + lane
                    prod = jnp.where(col < NWORDS, prod, 0.0)
                accs[k % 8] = accs[k % 8] + prod
            while len(accs) > 1:
                accs = [a + b for a, b in zip(accs[::2], accs[1::2])]
            return accs[0]

        @pl.when(i < NBLK - 1)
        def _():
            acc_ref[...] += chunk_sum(None)

        @pl.when(i == NBLK - 1)
        def _():
            part = chunk_sum((NWORDS - (NBLK - 1) * C) // 128)
            score = jnp.sum(acc_ref[...] + part, axis=1)[None, :] + b_ref[...]
            m = jnp.max(score, axis=1, keepdims=True)
            lse = jnp.log(jnp.sum(jnp.exp(score - m), axis=1, keepdims=True)) + m
            o_ref[...] = score - lse

    return pl.pallas_call(
        body,
        grid=(NBLK,),
        in_specs=[
            pl.BlockSpec((D, C), lambda i: (0, i)),
            pl.BlockSpec((C,), lambda i: (i,)),
            pl.BlockSpec((C,), lambda i: (i + HB // C)),
            pl.BlockSpec((1, D), lambda i: (0, 0)),
        ],
        out_specs=pl.BlockSpec((1, D), lambda i: (0, 0)),
        out_shape=jax.ShapeDtypeStruct((1, D), jnp.float32),
        scratch_shapes=[pltpu.VMEM((D, 128), jnp.float32)],
    )(emb_t, hist, hist, bias2d)


def kernel(inputs, embeddings, bias):
    idx = inputs.astype(jnp.int32)
    emb_t = embeddings.T
    hist = _sc_histogram(idx)
    return _tc_matvec_finalize(emb_t, hist, bias.reshape(1, D))


# speedup vs baseline: 1.1091x; 1.0352x over previous
import functools

import jax
import jax.numpy as jnp
from jax import lax
from jax.experimental import pallas as pl
from jax.experimental.pallas import tpu as pltpu
from jax.experimental.pallas import tpu_sc as plsc

NC = 2
NS = 16
NW = NC * NS
SEQ = 16384
D = 16
NWORDS = 1000000
PER_W = SEQ // NS
HB = 1 << 20
STRIPE = HB // NS
ZCH = 4096
CHUNK = 128
NCHUNK = PER_W // CHUNK

C = 65536
NBLK = (NWORDS + C - 1) // C


def _sc_histogram(idx):
    mesh = plsc.VectorSubcoreMesh(
        core_axis_name="c", subcore_axis_name="s", num_cores=1
    )

    @functools.partial(
        pl.kernel,
        mesh=mesh,
        out_type=jax.ShapeDtypeStruct((HB,), jnp.float32),
        scratch_types=[
            pltpu.VMEM((NCHUNK, CHUNK), jnp.int32),
            pltpu.VMEM((CHUNK,), jnp.float32),
            pltpu.VMEM((ZCH,), jnp.float32),
            pltpu.VMEM_SHARED((HB,), jnp.float32),
            pltpu.SemaphoreType.DMA,
        ],
    )
    def body(idx_hbm, out_hbm, idx2_v, ones_v, z_v, hsh, sem):
        s = lax.axis_index("s")
        wid = s
        for k in range(NCHUNK):
            pltpu.sync_copy(
                idx_hbm.at[pl.ds(wid * PER_W + k * CHUNK, CHUNK)], idx2_v.at[k]
            )

        one = jnp.ones((16,), jnp.float32)
        zero = jnp.zeros((16,), jnp.float32)
        for z in range(ZCH // 16):
            z_v[pl.ds(z * 16, 16)] = zero
        for z in range(CHUNK // 16):
            ones_v[pl.ds(z * 16, 16)] = one
        for z in range(STRIPE // ZCH):
            pltpu.sync_copy(z_v, hsh.at[pl.ds(s * STRIPE + z * ZCH, ZCH)])

        plsc.subcore_barrier()
        for k in range(NCHUNK):
            pltpu.sync_copy(ones_v, hsh.at[idx2_v.at[k]], add=True)
        plsc.subcore_barrier()

        pltpu.sync_copy(
            hsh.at[pl.ds(s * STRIPE, STRIPE)],
            out_hbm.at[pl.ds(s * STRIPE, STRIPE)],
        )

    return body(idx)


def _tc_matvec_finalize(emb_t, hist, bias2d):
    def body(t_ref, h_ref, b_ref, o_ref, acc_ref):
        i = pl.program_id(0)

        @pl.when(i == 0)
        def _():
            acc_ref[...] = jnp.zeros_like(acc_ref)

        def chunk_sum(mask_from):
            accs = [jnp.zeros((D, 128), jnp.float32) for _ in range(8)]
            lane = lax.broadcasted_iota(jnp.int32, (1, 128), 1)
            for k in range(C // 128):
                sl = pl.ds(k * 128, 128)
                prod = t_ref[:, sl] * h_ref[sl][None, :]
                if mask_from is not None and k >= mask_from:
                    col = i * C + k * 128 + lane
                    prod = jnp.where(col < NWORDS, prod, 0.0)
                accs[k % 8] = accs[k % 8] + prod
            while len(accs) > 1:
                accs = [a + b for a, b in zip(accs[::2], accs[1::2])]
            return accs[0]

        @pl.when(i < NBLK - 1)
        def _():
            acc_ref[...] += chunk_sum(None)

        @pl.when(i == NBLK - 1)
        def _():
            part = chunk_sum((NWORDS - (NBLK - 1) * C) // 128)
            score = jnp.sum(acc_ref[...] + part, axis=1)[None, :] + b_ref[...]
            m = jnp.max(score, axis=1, keepdims=True)
            lse = jnp.log(jnp.sum(jnp.exp(score - m), axis=1, keepdims=True)) + m
            o_ref[...] = score - lse

    return pl.pallas_call(
        body,
        grid=(NBLK,),
        in_specs=[
            pl.BlockSpec((D, C), lambda i: (0, i)),
            pl.BlockSpec((C,), lambda i: (i,)),
            pl.BlockSpec((1, D), lambda i: (0, 0)),
        ],
        out_specs=pl.BlockSpec((1, D), lambda i: (0, 0)),
        out_shape=jax.ShapeDtypeStruct((1, D), jnp.float32),
        scratch_shapes=[pltpu.VMEM((D, 128), jnp.float32)],
    )(emb_t, hist, bias2d)


def kernel(inputs, embeddings, bias):
    idx = inputs.astype(jnp.int32)
    emb_t = embeddings.T
    hist = _sc_histogram(idx)
    return _tc_matvec_finalize(emb_t, hist, bias.reshape(1, D))


# speedup vs baseline: 1.1401x; 1.0280x over previous
import functools

import jax
import jax.numpy as jnp
from jax import lax
from jax.experimental import pallas as pl
from jax.experimental.pallas import tpu as pltpu
from jax.experimental.pallas import tpu_sc as plsc

NC = 2
NS = 16
NW = NC * NS
SEQ = 16384
D = 16
NWORDS = 1000000
PER_W = SEQ // NS
HB = 1 << 20
STRIPE = HB // NS
ZCH = 4096
CHUNK = 128
NCHUNK = PER_W // CHUNK

C = 131072
NBLK = (NWORDS + C - 1) // C


def _sc_histogram(idx):
    mesh = plsc.VectorSubcoreMesh(
        core_axis_name="c", subcore_axis_name="s", num_cores=1
    )

    @functools.partial(
        pl.kernel,
        mesh=mesh,
        out_type=jax.ShapeDtypeStruct((HB,), jnp.float32),
        scratch_types=[
            pltpu.VMEM((NCHUNK, CHUNK), jnp.int32),
            pltpu.VMEM((CHUNK,), jnp.float32),
            pltpu.VMEM((ZCH,), jnp.float32),
            pltpu.VMEM_SHARED((HB,), jnp.float32),
            pltpu.SemaphoreType.DMA,
        ],
    )
    def body(idx_hbm, out_hbm, idx2_v, ones_v, z_v, hsh, sem):
        s = lax.axis_index("s")
        wid = s
        for k in range(NCHUNK):
            pltpu.sync_copy(
                idx_hbm.at[pl.ds(wid * PER_W + k * CHUNK, CHUNK)], idx2_v.at[k]
            )

        one = jnp.ones((16,), jnp.float32)
        zero = jnp.zeros((16,), jnp.float32)
        for z in range(ZCH // 16):
            z_v[pl.ds(z * 16, 16)] = zero
        for z in range(CHUNK // 16):
            ones_v[pl.ds(z * 16, 16)] = one
        for z in range(STRIPE // ZCH):
            pltpu.sync_copy(z_v, hsh.at[pl.ds(s * STRIPE + z * ZCH, ZCH)])

        plsc.subcore_barrier()
        for k in range(NCHUNK):
            pltpu.sync_copy(ones_v, hsh.at[idx2_v.at[k]], add=True)
        plsc.subcore_barrier()

        pltpu.sync_copy(
            hsh.at[pl.ds(s * STRIPE, STRIPE)],
            out_hbm.at[pl.ds(s * STRIPE, STRIPE)],
        )

    return body(idx)


def _tc_matvec_finalize(emb_t, hist, bias2d):
    def body(t_ref, h_ref, b_ref, o_ref, acc_ref):
        i = pl.program_id(0)

        @pl.when(i == 0)
        def _():
            acc_ref[...] = jnp.zeros_like(acc_ref)

        def chunk_sum(mask_from):
            accs = [jnp.zeros((D, 128), jnp.float32) for _ in range(8)]
            lane = lax.broadcasted_iota(jnp.int32, (1, 128), 1)
            for k in range(C // 128):
                sl = pl.ds(k * 128, 128)
                prod = t_ref[:, sl] * h_ref[sl][None, :]
                if mask_from is not None and k >= mask_from:
                    col = i * C + k * 128 + lane
                    prod = jnp.where(col < NWORDS, prod, 0.0)
                accs[k % 8] = accs[k % 8] + prod
            while len(accs) > 1:
                accs = [a + b for a, b in zip(accs[::2], accs[1::2])]
            return accs[0]

        @pl.when(i < NBLK - 1)
        def _():
            acc_ref[...] += chunk_sum(None)

        @pl.when(i == NBLK - 1)
        def _():
            part = chunk_sum((NWORDS - (NBLK - 1) * C) // 128)
            score = jnp.sum(acc_ref[...] + part, axis=1)[None, :] + b_ref[...]
            m = jnp.max(score, axis=1, keepdims=True)
            lse = jnp.log(jnp.sum(jnp.exp(score - m), axis=1, keepdims=True)) + m
            o_ref[...] = score - lse

    return pl.pallas_call(
        body,
        grid=(NBLK,),
        in_specs=[
            pl.BlockSpec((D, C), lambda i: (0, i)),
            pl.BlockSpec((C,), lambda i: (i,)),
            pl.BlockSpec((1, D), lambda i: (0, 0)),
        ],
        out_specs=pl.BlockSpec((1, D), lambda i: (0, 0)),
        out_shape=jax.ShapeDtypeStruct((1, D), jnp.float32),
        scratch_shapes=[pltpu.VMEM((D, 128), jnp.float32)],
    )(emb_t, hist, bias2d)


def kernel(inputs, embeddings, bias):
    idx = inputs.astype(jnp.int32)
    emb_t = embeddings.T
    hist = _sc_histogram(idx)
    return _tc_matvec_finalize(emb_t, hist, bias.reshape(1, D))
